# baseline (speedup 1.0000x reference)
import functools

import jax
import jax.numpy as jnp
from jax import lax
from jax.experimental import pallas as pl
from jax.experimental.pallas import tpu as pltpu
from jax.experimental.pallas import tpu_sc as plsc

_N = 10000
_E = 320000
_D = 128
_H = 8
_C = 16

_NC = 2
_NS = 16
_NW = _NC * _NS
_EPT = _E // _NW
_B = 40
_NBLK = _EPT // _B
_G = _B // 16
_NP = 10240
_RPT = _NP // _NS


def _proj_body(x_ref, wl_ref, bl_ref, wr_ref, br_ref, t_ref):
    x = x_ref[...]
    dn = (((1,), (1,)), ((), ()))
    t_ref[0] = lax.dot_general(x, wl_ref[...], dn,
                               preferred_element_type=jnp.float32) + bl_ref[...]
    t_ref[1] = lax.dot_general(x, wr_ref[...], dn,
                               preferred_element_type=jnp.float32) + br_ref[...]


def _project(x, wl, bl, wr, br):
    bn = 1000
    grid = _N // bn
    return pl.pallas_call(
        _proj_body,
        grid=(grid,),
        in_specs=[
            pl.BlockSpec((bn, _D), lambda i: (i, 0)),
            pl.BlockSpec((_D, _D), lambda i: (0, 0)),
            pl.BlockSpec((1, _D), lambda i: (0, 0)),
            pl.BlockSpec((_D, _D), lambda i: (0, 0)),
            pl.BlockSpec((1, _D), lambda i: (0, 0)),
        ],
        out_specs=pl.BlockSpec((2, bn, _D), lambda i: (0, i, 0)),
        out_shape=jax.ShapeDtypeStruct((2, _N, _D), jnp.float32),
    )(x, wl, bl, wr, br)


def _sc_edge_body(t_hbm, ei_hbm, attb_hbm, zacc_hbm, zden_hbm,
                  acc_out, den_out,
                  idx_b, dst_s, rows_v, msg_v, den_v, att_v,
                  acc_sh, den_sh, s_idx, s_gat, s_sct, s_dst):
    cid = lax.axis_index("c")
    sid = lax.axis_index("s")
    wid = sid * _NC + cid
    tbase = wid * _EPT * 2

    lane = lax.iota(jnp.int32, 16)
    last = jnp.full((16,), 15, jnp.int32)
    att_rows = [None] * _H

    def start_idx(p, blk):
        base = tbase + blk * 2 * _B
        pltpu.make_async_copy(ei_hbm.at[pl.ds(base, 2 * _B)],
                              idx_b[p].at[pl.ds(0, 2 * _B)], s_idx[p]).start()

    def wait_idx(p):
        pltpu.make_async_copy(ei_hbm.at[pl.ds(0, 2 * _B)],
                              idx_b[p].at[pl.ds(0, 2 * _B)], s_idx[p]).wait()

    def fixup_idx(p):
        for k in range(_B // 16 + 1):
            off = _B + 16 * k
            idx_b[p][pl.ds(off, 16)] = idx_b[p][pl.ds(off, 16)] + _N

    def start_gather(p):
        pltpu.make_async_copy(t_hbm.at[idx_b[p].at[pl.ds(0, 2 * _B)]],
                              rows_v[p], s_gat[p]).start()

    def wait_gather(p):
        pltpu.make_async_copy(t_hbm.at[idx_b[p].at[pl.ds(0, 2 * _B)]],
                              rows_v[p], s_gat[p]).wait()

    def start_dsts(p, blk):
        base = tbase + blk * 2 * _B + _B
        pltpu.make_async_copy(ei_hbm.at[pl.ds(base, _B)], dst_s[p], s_dst[p]).start()

    def wait_dsts(p):
        pltpu.make_async_copy(ei_hbm.at[pl.ds(0, _B)], dst_s[p], s_dst[p]).wait()

    def start_scatter(p):
        pltpu.async_copy(msg_v[p], acc_sh.at[dst_s[p]], s_sct[p], add=True)
        pltpu.async_copy(den_v[p], den_sh.at[dst_s[p]], s_sct[p], add=True)

    def wait_scatter(p):
        pltpu.make_async_copy(msg_v[p], acc_sh.at[pl.ds(0, _B)], s_sct[p]).wait()
        pltpu.make_async_copy(den_v[p], den_sh.at[pl.ds(0, _B)], s_sct[p]).wait()

    def compute(p):
        rows_b, msg_b, den_b = rows_v[p], msg_v[p], den_v[p]

        @plsc.parallel_loop(0, _B, 1, unroll=2)
        def edge_body(i):
            ws = []
            avs = []
            den_row = jnp.zeros((16,), jnp.float32)
            for h in range(_H):
                a = rows_b[i, pl.ds(h * _C, _C)]
                b = rows_b[_B + i, pl.ds(h * _C, _C)]
                s = a + b
                e = jnp.where(s > 0, s, 0.2 * s)
                cs = plsc.cumsum(e * att_rows[h])
                logit = cs.at[last].get(mode="promise_in_bounds")
                w = jnp.exp(logit)
                ws.append(w)
                avs.append(a)
                den_row = jnp.where(lane == h, w, den_row)
            den_b[i, :] = den_row
            for h in range(_H):
                msg_b[i, pl.ds(h * _C, _C)] = ws[h] * avs[h]

    start_idx(0, 0)
    start_idx(1, 1)

    pltpu.sync_copy(attb_hbm, att_v)
    for h in range(_H):
        att_rows[h] = att_v[h, :]
    rows0 = pl.ds(sid * _RPT, _RPT)
    pltpu.sync_copy(zacc_hbm.at[rows0], acc_sh.at[rows0])
    pltpu.sync_copy(zden_hbm.at[rows0], den_sh.at[rows0])

    wait_idx(0)
    fixup_idx(0)
    start_gather(0)
    plsc.subcore_barrier()

    def head_step(p, blk):
        wait_idx(1 - p)
        fixup_idx(1 - p)
        start_gather(1 - p)
        wait_gather(p)
        start_idx(p, blk + 2)
        start_dsts(p, blk)
        compute(p)
        wait_dsts(p)
        start_scatter(p)

    head_step(0, 0)
    head_step(1, 1)

    def steady_step(p, blk):
        wait_idx(1 - p)
        fixup_idx(1 - p)
        start_gather(1 - p)
        wait_gather(p)
        start_idx(p, blk + 2)
        wait_scatter(p)
        start_dsts(p, blk)
        compute(p)
        wait_dsts(p)
        start_scatter(p)

    def super_body(j, _):
        steady_step(0, 2 * j)
        steady_step(1, 2 * j + 1)
        return ()

    lax.fori_loop(1, _NBLK // 2 - 1, super_body, ())

    wait_idx(1)
    fixup_idx(1)
    start_gather(1)
    wait_gather(0)
    wait_scatter(0)
    start_dsts(0, _NBLK - 2)
    compute(0)
    wait_dsts(0)
    start_scatter(0)

    wait_gather(1)
    wait_scatter(1)
    start_dsts(1, _NBLK - 1)
    compute(1)
    wait_dsts(1)
    start_scatter(1)

    wait_scatter(0)
    wait_scatter(1)

    plsc.subcore_barrier()

    pltpu.sync_copy(acc_sh.at[rows0], acc_out.at[cid, rows0])
    pltpu.sync_copy(den_sh.at[rows0], den_out.at[cid, rows0])


def _sc_edge(t2, ei2, attb, zacc, zden):
    mesh = plsc.VectorSubcoreMesh(core_axis_name="c", subcore_axis_name="s")
    f = pl.kernel(
        _sc_edge_body,
        out_type=(
            jax.ShapeDtypeStruct((_NC, _NP, _D), jnp.float32),
            jax.ShapeDtypeStruct((_NC, _NP, 16), jnp.float32),
        ),
        mesh=mesh,
        compiler_params=pltpu.CompilerParams(needs_layout_passes=False, use_tc_tiling_on_sc=False),
        scratch_types=[
            [pltpu.VMEM((2 * _B + 16,), jnp.int32)] * 2,
            [pltpu.VMEM((_B,), jnp.int32)] * 2,
            [pltpu.VMEM((2 * _B, _D), jnp.float32)] * 2,
            [pltpu.VMEM((_B, _D), jnp.float32)] * 2,
            [pltpu.VMEM((_B, 16), jnp.float32)] * 2,
            pltpu.VMEM((_H, _C), jnp.float32),
            pltpu.VMEM_SHARED((_NP, _D), jnp.float32),
            pltpu.VMEM_SHARED((_NP, 16), jnp.float32),
            [pltpu.SemaphoreType.DMA] * 2,
            [pltpu.SemaphoreType.DMA] * 2,
            [pltpu.SemaphoreType.DMA] * 2,
            [pltpu.SemaphoreType.DMA] * 2,
        ],
    )
    return f(t2, ei2, attb, zacc, zden)


def _fin_body(acc_ref, den_ref, x_ref, bias_ref, o_ref):
    agg = acc_ref[0] + acc_ref[1]
    den = den_ref[0] + den_ref[1]
    row_id = lax.broadcasted_iota(jnp.int32, (16, _D), 0)
    col_id = lax.broadcasted_iota(jnp.int32, (16, _D), 1)
    k = (col_id // _C == row_id).astype(jnp.float32)
    den_exp = jnp.dot(den, k, preferred_element_type=jnp.float32)
    z = agg / (den_exp + 1e-16) + bias_ref[...] + x_ref[...]
    o_ref[...] = 0.5 * z * (1.0 + lax.erf(z * 0.7071067811865476))


def _finalize(acc, den, x, bias):
    bn = 1000
    grid = _N // bn
    return pl.pallas_call(
        _fin_body,
        grid=(grid,),
        in_specs=[
            pl.BlockSpec((_NC, bn, _D), lambda i: (0, i, 0)),
            pl.BlockSpec((_NC, bn, 16), lambda i: (0, i, 0)),
            pl.BlockSpec((bn, _D), lambda i: (i, 0)),
            pl.BlockSpec((1, _D), lambda i: (0, 0)),
        ],
        out_specs=pl.BlockSpec((bn, _D), lambda i: (i, 0)),
        out_shape=jax.ShapeDtypeStruct((_N, _D), jnp.float32),
    )(acc, den, x, bias)


@jax.jit
def kernel(x, edge_index, Wl, bl, Wr, br, att, bias):
    t = _project(x, Wl, bl.reshape(1, _D), Wr, br.reshape(1, _D))
    t2 = t.reshape(2 * _N, _D)
    ei2 = (edge_index.astype(jnp.int32)
           .reshape(2, _E // _B, _B)
           .transpose(1, 0, 2)
           .reshape(2 * _E))
    zacc = jnp.zeros((_NP, _D), jnp.float32)
    zden = jnp.zeros((_NP, 16), jnp.float32)
    acc, den = _sc_edge(t2, ei2, att, zacc, zden)
    return _finalize(acc, den, x, bias.reshape(1, _D))


# speedup vs baseline: 1.1854x; 1.1854x over previous
import functools

import jax
import jax.numpy as jnp
from jax import lax
from jax.experimental import pallas as pl
from jax.experimental.pallas import tpu as pltpu
from jax.experimental.pallas import tpu_sc as plsc

_N = 10000
_E = 320000
_D = 128
_H = 8
_C = 16

_NC = 2
_NS = 16
_NW = _NC * _NS
_EPT = _E // _NW
_B = 40
_NBLK = _EPT // _B
_G = _B // 16
_NP = 10240
_RPT = _NP // _NS


def _proj_body(x_ref, wl_ref, bl_ref, wr_ref, br_ref, xl_ref, xr_ref):
    x = x_ref[...]
    dn = (((1,), (1,)), ((), ()))
    xl_ref[...] = lax.dot_general(x, wl_ref[...], dn,
                                  preferred_element_type=jnp.float32) + bl_ref[...]
    xr_ref[...] = lax.dot_general(x, wr_ref[...], dn,
                                  preferred_element_type=jnp.float32) + br_ref[...]


def _project(x, wl, bl, wr, br):
    bn = 1000
    grid = _N // bn
    return pl.pallas_call(
        _proj_body,
        grid=(grid,),
        in_specs=[
            pl.BlockSpec((bn, _D), lambda i: (i, 0)),
            pl.BlockSpec((_D, _D), lambda i: (0, 0)),
            pl.BlockSpec((1, _D), lambda i: (0, 0)),
            pl.BlockSpec((_D, _D), lambda i: (0, 0)),
            pl.BlockSpec((1, _D), lambda i: (0, 0)),
        ],
        out_specs=[
            pl.BlockSpec((bn, _D), lambda i: (i, 0)),
            pl.BlockSpec((bn, _D), lambda i: (i, 0)),
        ],
        out_shape=[
            jax.ShapeDtypeStruct((_N, _D), jnp.float32),
            jax.ShapeDtypeStruct((_N, _D), jnp.float32),
        ],
    )(x, wl, bl, wr, br)


def _sc_edge_body(xl_hbm, xr_hbm, ei_hbm, attb_hbm, zacc_hbm, zden_hbm,
                  acc_out, den_out,
                  src_i, dst_i, dst_s, xl_r, xr_r, msg_v, den_v, att_v,
                  acc_sh, den_sh, s_idx, s_gat, s_sct, s_dst):
    cid = lax.axis_index("c")
    sid = lax.axis_index("s")
    wid = sid * _NC + cid
    tbase = wid * _EPT

    lane = lax.iota(jnp.int32, 16)
    last = jnp.full((16,), 15, jnp.int32)
    att_rows = [None] * _H

    def start_idx(p, blk):
        base = tbase + blk * _B
        pltpu.make_async_copy(ei_hbm.at[pl.ds(base, _B)], src_i[p], s_idx[p]).start()
        pltpu.make_async_copy(ei_hbm.at[pl.ds(_E + base, _B)], dst_i[p], s_idx[p]).start()

    def wait_idx(p):
        pltpu.make_async_copy(ei_hbm.at[pl.ds(0, _B)], src_i[p], s_idx[p]).wait()
        pltpu.make_async_copy(ei_hbm.at[pl.ds(0, _B)], dst_i[p], s_idx[p]).wait()

    def start_gather(p):
        pltpu.make_async_copy(xl_hbm.at[src_i[p]], xl_r[p], s_gat[p]).start()
        pltpu.make_async_copy(xr_hbm.at[dst_i[p]], xr_r[p], s_gat[p]).start()

    def wait_gather(p):
        pltpu.make_async_copy(xl_hbm.at[src_i[p]], xl_r[p], s_gat[p]).wait()
        pltpu.make_async_copy(xr_hbm.at[dst_i[p]], xr_r[p], s_gat[p]).wait()

    def start_dsts(p, blk):
        base = _E + tbase + blk * _B
        pltpu.make_async_copy(ei_hbm.at[pl.ds(base, _B)], dst_s[p], s_dst[p]).start()

    def wait_dsts(p):
        pltpu.make_async_copy(ei_hbm.at[pl.ds(0, _B)], dst_s[p], s_dst[p]).wait()

    def start_scatter(p):
        pltpu.async_copy(msg_v[p], acc_sh.at[dst_s[p]], s_sct[p], add=True)
        pltpu.async_copy(den_v[p], den_sh.at[dst_s[p]], s_sct[p], add=True)

    def wait_scatter(p):
        pltpu.make_async_copy(msg_v[p], acc_sh.at[pl.ds(0, _B)], s_sct[p]).wait()
        pltpu.make_async_copy(den_v[p], den_sh.at[pl.ds(0, _B)], s_sct[p]).wait()

    def compute(p):
        xl_b, xr_b, msg_b, den_b = xl_r[p], xr_r[p], msg_v[p], den_v[p]

        @plsc.parallel_loop(0, _B, 1, unroll=2)
        def edge_body(i):
            ws = []
            avs = []
            den_row = jnp.zeros((16,), jnp.float32)
            for h in range(_H):
                a = xl_b[i, pl.ds(h * _C, _C)]
                b = xr_b[i, pl.ds(h * _C, _C)]
                s = a + b
                e = jnp.where(s > 0, s, 0.2 * s)
                cs = plsc.cumsum(e * att_rows[h])
                logit = cs.at[last].get(mode="promise_in_bounds")
                w = jnp.exp(logit)
                ws.append(w)
                avs.append(a)
                den_row = jnp.where(lane == h, w, den_row)
            den_b[i, :] = den_row
            for h in range(_H):
                msg_b[i, pl.ds(h * _C, _C)] = ws[h] * avs[h]

    start_idx(0, 0)
    start_idx(1, 1)

    pltpu.sync_copy(attb_hbm, att_v)
    for h in range(_H):
        att_rows[h] = att_v[h, :]
    rows0 = pl.ds(sid * _RPT, _RPT)
    pltpu.sync_copy(zacc_hbm.at[rows0], acc_sh.at[rows0])
    pltpu.sync_copy(zden_hbm.at[rows0], den_sh.at[rows0])

    wait_idx(0)
    start_gather(0)
    plsc.subcore_barrier()

    def head_step(p, blk):
        wait_idx(1 - p)
        start_gather(1 - p)
        wait_gather(p)
        start_idx(p, blk + 2)
        start_dsts(p, blk)
        compute(p)
        wait_dsts(p)
        start_scatter(p)

    head_step(0, 0)
    head_step(1, 1)

    def steady_step(p, blk):
        wait_idx(1 - p)
        start_gather(1 - p)
        wait_scatter(p)
        start_dsts(p, blk)
        wait_gather(p)
        start_idx(p, blk + 2)
        compute(p)
        wait_dsts(p)
        start_scatter(p)

    def super_body(j, _):
        steady_step(0, 2 * j)
        steady_step(1, 2 * j + 1)
        return ()

    lax.fori_loop(1, _NBLK // 2 - 1, super_body, ())

    wait_idx(1)
    start_gather(1)
    wait_gather(0)
    wait_scatter(0)
    start_dsts(0, _NBLK - 2)
    compute(0)
    wait_dsts(0)
    start_scatter(0)

    wait_gather(1)
    wait_scatter(1)
    start_dsts(1, _NBLK - 1)
    compute(1)
    wait_dsts(1)
    start_scatter(1)

    wait_scatter(0)
    wait_scatter(1)

    plsc.subcore_barrier()

    pltpu.sync_copy(acc_sh.at[rows0], acc_out.at[cid, rows0])
    pltpu.sync_copy(den_sh.at[rows0], den_out.at[cid, rows0])


def _sc_edge(xl, xr, ei_flat, attb, zacc, zden):
    mesh = plsc.VectorSubcoreMesh(core_axis_name="c", subcore_axis_name="s")
    f = pl.kernel(
        _sc_edge_body,
        out_type=(
            jax.ShapeDtypeStruct((_NC, _NP, _D), jnp.float32),
            jax.ShapeDtypeStruct((_NC, _NP, 16), jnp.float32),
        ),
        mesh=mesh,
        compiler_params=pltpu.CompilerParams(needs_layout_passes=False, use_tc_tiling_on_sc=False),
        scratch_types=[
            [pltpu.VMEM((_B,), jnp.int32)] * 2,
            [pltpu.VMEM((_B,), jnp.int32)] * 2,
            [pltpu.VMEM((_B,), jnp.int32)] * 2,
            [pltpu.VMEM((_B, _D), jnp.float32)] * 2,
            [pltpu.VMEM((_B, _D), jnp.float32)] * 2,
            [pltpu.VMEM((_B, _D), jnp.float32)] * 2,
            [pltpu.VMEM((_B, 16), jnp.float32)] * 2,
            pltpu.VMEM((_H, _C), jnp.float32),
            pltpu.VMEM_SHARED((_NP, _D), jnp.float32),
            pltpu.VMEM_SHARED((_NP, 16), jnp.float32),
            [pltpu.SemaphoreType.DMA] * 2,
            [pltpu.SemaphoreType.DMA] * 2,
            [pltpu.SemaphoreType.DMA] * 2,
            [pltpu.SemaphoreType.DMA] * 2,
        ],
    )
    return f(xl, xr, ei_flat, attb, zacc, zden)


def _fin_body(acc_ref, den_ref, x_ref, bias_ref, o_ref):
    agg = acc_ref[0] + acc_ref[1]
    den = den_ref[0] + den_ref[1]
    row_id = lax.broadcasted_iota(jnp.int32, (16, _D), 0)
    col_id = lax.broadcasted_iota(jnp.int32, (16, _D), 1)
    k = (col_id // _C == row_id).astype(jnp.float32)
    den_exp = jnp.dot(den, k, preferred_element_type=jnp.float32)
    z = agg / (den_exp + 1e-16) + bias_ref[...] + x_ref[...]
    o_ref[...] = 0.5 * z * (1.0 + lax.erf(z * 0.7071067811865476))


def _finalize(acc, den, x, bias):
    bn = 1000
    grid = _N // bn
    return pl.pallas_call(
        _fin_body,
        grid=(grid,),
        in_specs=[
            pl.BlockSpec((_NC, bn, _D), lambda i: (0, i, 0)),
            pl.BlockSpec((_NC, bn, 16), lambda i: (0, i, 0)),
            pl.BlockSpec((bn, _D), lambda i: (i, 0)),
            pl.BlockSpec((1, _D), lambda i: (0, 0)),
        ],
        out_specs=pl.BlockSpec((bn, _D), lambda i: (i, 0)),
        out_shape=jax.ShapeDtypeStruct((_N, _D), jnp.float32),
    )(acc, den, x, bias)


@jax.jit
def kernel(x, edge_index, Wl, bl, Wr, br, att, bias):
    xl, xr = _project(x, Wl, bl.reshape(1, _D), Wr, br.reshape(1, _D))
    ei_flat = edge_index.astype(jnp.int32).reshape(2 * _E)
    zacc = jnp.zeros((_NP, _D), jnp.float32)
    zden = jnp.zeros((_NP, 16), jnp.float32)
    acc, den = _sc_edge(xl, xr, ei_flat, att, zacc, zden)
    return _finalize(acc, den, x, bias.reshape(1, _D))


# speedup vs baseline: 1.1885x; 1.0026x over previous
import jax
import jax.numpy as jnp
from jax import lax
from jax.experimental import pallas as pl
from jax.experimental.pallas import tpu as pltpu
from jax.experimental.pallas import tpu_sc as plsc

_N = 10000
_E = 320000
_D = 128
_H = 8
_C = 16

_NC = 2
_NS = 16
_NW = _NC * _NS
_EPT = _E // _NW
_B = 40
_NBLK = _EPT // _B
_NP = 10240
_RPT = _NP // _NS


def _proj_body(x_ref, wl_ref, bl_ref, wr_ref, br_ref, xl_ref, xr_ref):
    x = x_ref[...]
    dn = (((1,), (1,)), ((), ()))
    xl_ref[...] = lax.dot_general(x, wl_ref[...], dn,
                                  preferred_element_type=jnp.float32) + bl_ref[...]
    xr_ref[...] = lax.dot_general(x, wr_ref[...], dn,
                                  preferred_element_type=jnp.float32) + br_ref[...]


def _project(x, wl, bl, wr, br):
    bn = 1000
    grid = _N // bn
    return pl.pallas_call(
        _proj_body,
        grid=(grid,),
        in_specs=[
            pl.BlockSpec((bn, _D), lambda i: (i, 0)),
            pl.BlockSpec((_D, _D), lambda i: (0, 0)),
            pl.BlockSpec((1, _D), lambda i: (0, 0)),
            pl.BlockSpec((_D, _D), lambda i: (0, 0)),
            pl.BlockSpec((1, _D), lambda i: (0, 0)),
        ],
        out_specs=[
            pl.BlockSpec((bn, _D), lambda i: (i, 0)),
            pl.BlockSpec((bn, _D), lambda i: (i, 0)),
        ],
        out_shape=[
            jax.ShapeDtypeStruct((_N, _D), jnp.float32),
            jax.ShapeDtypeStruct((_N, _D), jnp.float32),
        ],
    )(x, wl, bl, wr, br)


def _sc_edge_body(xl_hbm, xr_hbm, ei_hbm, attb_hbm, zacc_hbm, zden_hbm,
                  acc_out, den_out,
                  src_i, dst_i, dst_s, xl_r, xr_r, msg_v, den_v, att_v,
                  acc_sh, den_sh, s_idx, s_gat, s_sct, s_dst):
    cid = lax.axis_index("c")
    sid = lax.axis_index("s")
    wid = sid * _NC + cid
    tbase = wid * _EPT

    lane = lax.iota(jnp.int32, 16)
    last = jnp.full((16,), 15, jnp.int32)
    att_rows = [None] * _H

    def start_idx(p, blk):
        base = tbase + blk * _B
        pltpu.make_async_copy(ei_hbm.at[pl.ds(base, _B)], src_i[p], s_idx[p]).start()
        pltpu.make_async_copy(ei_hbm.at[pl.ds(_E + base, _B)], dst_i[p], s_idx[p]).start()

    def wait_idx(p):
        pltpu.make_async_copy(ei_hbm.at[pl.ds(0, _B)], src_i[p], s_idx[p]).wait()
        pltpu.make_async_copy(ei_hbm.at[pl.ds(0, _B)], dst_i[p], s_idx[p]).wait()

    def start_gather(p):
        pltpu.make_async_copy(xl_hbm.at[src_i[p]], xl_r[p], s_gat[p]).start()
        pltpu.make_async_copy(xr_hbm.at[dst_i[p]], xr_r[p], s_gat[p]).start()

    def wait_gather(p):
        pltpu.make_async_copy(xl_hbm.at[src_i[p]], xl_r[p], s_gat[p]).wait()
        pltpu.make_async_copy(xr_hbm.at[dst_i[p]], xr_r[p], s_gat[p]).wait()

    def start_dsts(p, blk):
        base = _E + tbase + blk * _B
        pltpu.make_async_copy(ei_hbm.at[pl.ds(base, _B)], dst_s[p], s_dst[p]).start()

    def wait_dsts(p):
        pltpu.make_async_copy(ei_hbm.at[pl.ds(0, _B)], dst_s[p], s_dst[p]).wait()

    def start_scatter(p):
        pltpu.async_copy(msg_v[p], acc_sh.at[dst_s[p]], s_sct[p], add=True)
        pltpu.async_copy(den_v[p], den_sh.at[dst_s[p]], s_sct[p], add=True)

    def wait_scatter(p):
        pltpu.make_async_copy(msg_v[p], acc_sh.at[pl.ds(0, _B)], s_sct[p]).wait()
        pltpu.make_async_copy(den_v[p], den_sh.at[pl.ds(0, _B)], s_sct[p]).wait()

    def compute(p):
        xl_b, xr_b, msg_b, den_b = xl_r[p], xr_r[p], msg_v[p], den_v[p]

        @plsc.parallel_loop(0, _B, 1, unroll=2)
        def edge_body(i):
            ws = []
            avs = []
            den_row = jnp.zeros((16,), jnp.float32)
            for h in range(_H):
                a = xl_b[i, pl.ds(h * _C, _C)]
                b = xr_b[i, pl.ds(h * _C, _C)]
                s = a + b
                e = jnp.where(s > 0, s, 0.2 * s)
                cs = plsc.cumsum(e * att_rows[h])
                logit = cs.at[last].get(mode="promise_in_bounds")
                w = jnp.exp(logit)
                ws.append(w)
                avs.append(a)
                den_row = jnp.where(lane == h, w, den_row)
            den_b[i, :] = den_row
            for h in range(_H):
                msg_b[i, pl.ds(h * _C, _C)] = ws[h] * avs[h]

    start_idx(0, 0)
    start_idx(1, 1)

    pltpu.sync_copy(attb_hbm, att_v)
    for h in range(_H):
        att_rows[h] = att_v[h, :]
    rows0 = pl.ds(sid * _RPT, _RPT)
    pltpu.sync_copy(zacc_hbm.at[rows0], acc_sh.at[rows0])
    pltpu.sync_copy(zden_hbm.at[rows0], den_sh.at[rows0])

    wait_idx(0)
    start_gather(0)
    plsc.subcore_barrier()

    def head_step(p, blk):
        wait_idx(1 - p)
        start_gather(1 - p)
        wait_gather(p)
        start_idx(p, blk + 2)
        start_dsts(p, blk)
        compute(p)
        wait_dsts(p)
        start_scatter(p)

    head_step(0, 0)
    head_step(1, 1)

    def steady_step(p, blk):
        wait_idx(1 - p)
        start_gather(1 - p)
        wait_scatter(p)
        start_dsts(p, blk)
        wait_gather(p)
        start_idx(p, blk + 2)
        compute(p)
        wait_dsts(p)
        start_scatter(p)

    def super_body(j, _):
        steady_step(0, 2 * j)
        steady_step(1, 2 * j + 1)
        return ()

    lax.fori_loop(1, _NBLK // 2 - 1, super_body, ())

    wait_idx(1)
    start_gather(1)
    wait_gather(0)
    wait_scatter(0)
    start_dsts(0, _NBLK - 2)
    compute(0)
    wait_dsts(0)
    start_scatter(0)

    wait_gather(1)
    wait_scatter(1)
    start_dsts(1, _NBLK - 1)
    compute(1)
    wait_dsts(1)
    start_scatter(1)

    wait_scatter(0)
    wait_scatter(1)

    plsc.subcore_barrier()

    pltpu.sync_copy(acc_sh.at[rows0], acc_out.at[cid, rows0])
    pltpu.sync_copy(den_sh.at[rows0], den_out.at[cid, rows0])


def _sc_edge(xl, xr, ei_flat, attb, zacc, zden):
    mesh = plsc.VectorSubcoreMesh(core_axis_name="c", subcore_axis_name="s")
    f = pl.kernel(
        _sc_edge_body,
        out_type=(
            jax.ShapeDtypeStruct((_NC, _NP, _D), jnp.float32),
            jax.ShapeDtypeStruct((_NC, _NP, 16), jnp.float32),
        ),
        mesh=mesh,
        compiler_params=pltpu.CompilerParams(needs_layout_passes=False, use_tc_tiling_on_sc=False),
        scratch_types=[
            [pltpu.VMEM((_B,), jnp.int32)] * 2,
            [pltpu.VMEM((_B,), jnp.int32)] * 2,
            [pltpu.VMEM((_B,), jnp.int32)] * 2,
            [pltpu.VMEM((_B, _D), jnp.float32)] * 2,
            [pltpu.VMEM((_B, _D), jnp.float32)] * 2,
            [pltpu.VMEM((_B, _D), jnp.float32)] * 2,
            [pltpu.VMEM((_B, 16), jnp.float32)] * 2,
            pltpu.VMEM((_H, _C), jnp.float32),
            pltpu.VMEM_SHARED((_NP, _D), jnp.float32),
            pltpu.VMEM_SHARED((_NP, 16), jnp.float32),
            [pltpu.SemaphoreType.DMA] * 2,
            [pltpu.SemaphoreType.DMA] * 2,
            [pltpu.SemaphoreType.DMA] * 2,
            [pltpu.SemaphoreType.DMA] * 2,
        ],
    )
    return f(xl, xr, ei_flat, attb, zacc, zden)


def _fin_body(acc_ref, den_ref, x_ref, bias_ref, o_ref):
    agg = acc_ref[0] + acc_ref[1]
    den = den_ref[0] + den_ref[1]
    row_id = lax.broadcasted_iota(jnp.int32, (16, _D), 0)
    col_id = lax.broadcasted_iota(jnp.int32, (16, _D), 1)
    k = (col_id // _C == row_id).astype(jnp.float32)
    den_exp = jnp.dot(den, k, preferred_element_type=jnp.float32)
    z = agg / (den_exp + 1e-16) + bias_ref[...] + x_ref[...]
    o_ref[...] = 0.5 * z * (1.0 + lax.erf(z * 0.7071067811865476))


def _finalize(acc, den, x, bias):
    bn = 1000
    grid = _N // bn
    return pl.pallas_call(
        _fin_body,
        grid=(grid,),
        in_specs=[
            pl.BlockSpec((_NC, bn, _D), lambda i: (0, i, 0)),
            pl.BlockSpec((_NC, bn, 16), lambda i: (0, i, 0)),
            pl.BlockSpec((bn, _D), lambda i: (i, 0)),
            pl.BlockSpec((1, _D), lambda i: (0, 0)),
        ],
        out_specs=pl.BlockSpec((bn, _D), lambda i: (i, 0)),
        out_shape=jax.ShapeDtypeStruct((_N, _D), jnp.float32),
    )(acc, den, x, bias)


@jax.jit
def kernel(x, edge_index, Wl, bl, Wr, br, att, bias):
    xl, xr = _project(x, Wl, bl.reshape(1, _D), Wr, br.reshape(1, _D))
    ei_flat = edge_index.astype(jnp.int32).reshape(2 * _E)
    zacc = jnp.zeros((_NP, _D), jnp.float32)
    zden = jnp.zeros((_NP, 16), jnp.float32)
    acc, den = _sc_edge(xl, xr, ei_flat, att, zacc, zden)
    return _finalize(acc, den, x, bias.reshape(1, _D))
